# baseline (speedup 1.0000x reference)
import functools

import jax
import jax.numpy as jnp
from jax import lax
from jax.experimental import pallas as pl
from jax.experimental.pallas import tpu as pltpu
from jax.experimental.pallas import tpu_sc as plsc

VOCAB = 1000000
HIDDEN = 32
NUM_LAYERS = 3
BATCH = 16384
EMB_DIM = HIDDEN * NUM_LAYERS
PAD_DIM = 128

_INFO = plsc.get_sparse_core_info()
_NC = _INFO.num_cores
_NS = _INFO.num_subcores
_NW = _NC * _NS
_B_PER_W = BATCH // _NW
_HALF = _B_PER_W // 2


def _gather_body(idx_hbm, emb1_hbm, emb2_hbm, out1_hbm, out2_hbm,
                 idx_v, rows1_v, rows2_v, sem1, sem2):
    wid = lax.axis_index("s") * _NC + lax.axis_index("c")
    base = wid * _B_PER_W
    for half in range(2):
        cbase = base + half * _HALF
        pltpu.sync_copy(idx_hbm.at[pl.ds(cbase, _HALF)], idx_v)
        c1 = pltpu.async_copy(emb1_hbm.at[idx_v], rows1_v, sem1)
        c2 = pltpu.async_copy(emb2_hbm.at[idx_v], rows2_v, sem2)
        c1.wait()
        pltpu.sync_copy(rows1_v, out1_hbm.at[pl.ds(cbase, _HALF), :])
        c2.wait()
        pltpu.sync_copy(rows2_v, out2_hbm.at[pl.ds(cbase, _HALF), :])


_gather2 = functools.partial(
    pl.kernel,
    mesh=plsc.VectorSubcoreMesh(core_axis_name="c", subcore_axis_name="s"),
    out_type=(
        jax.ShapeDtypeStruct((BATCH, PAD_DIM), jnp.float32),
        jax.ShapeDtypeStruct((BATCH, PAD_DIM), jnp.float32),
    ),
    scratch_types=[
        pltpu.VMEM((_HALF,), jnp.int32),
        pltpu.VMEM((_HALF, PAD_DIM), jnp.float32),
        pltpu.VMEM((_HALF, PAD_DIM), jnp.float32),
        pltpu.SemaphoreType.DMA,
        pltpu.SemaphoreType.DMA,
    ],
)(_gather_body)


def kernel(x_input, emb1, emb2):
    idx = x_input.astype(jnp.int32)
    emb1p = jnp.pad(emb1, ((0, 0), (0, PAD_DIM - EMB_DIM)))
    emb2p = jnp.pad(emb2, ((0, 0), (0, PAD_DIM - EMB_DIM)))
    out1, out2 = _gather2(idx, emb1p, emb2p)
    hc = out1[:, :EMB_DIM].reshape(NUM_LAYERS, -1, HIDDEN)
    hx = out2[:, :EMB_DIM].reshape(NUM_LAYERS, -1, HIDDEN)
    return (hc, hx)


# speedup vs baseline: 2.3268x; 2.3268x over previous
import functools

import jax
import jax.numpy as jnp
from jax import lax
from jax.experimental import pallas as pl
from jax.experimental.pallas import tpu as pltpu
from jax.experimental.pallas import tpu_sc as plsc

VOCAB = 1000000
HIDDEN = 32
NUM_LAYERS = 3
BATCH = 16384
EMB_DIM = HIDDEN * NUM_LAYERS
PAD_DIM = 128

_INFO = plsc.get_sparse_core_info()
_NC = _INFO.num_cores
_NS = _INFO.num_subcores
_NW = _NC * _NS
_B_PER_W = BATCH // _NW
_HALF = _B_PER_W // 2

_PAD_BLK = 8000


def _pad_body(x_ref, o_ref):
    o_ref[:, :EMB_DIM] = x_ref[...]


_pad128 = functools.partial(
    pl.pallas_call,
    out_shape=jax.ShapeDtypeStruct((VOCAB, PAD_DIM), jnp.float32),
    grid=(VOCAB // _PAD_BLK,),
    in_specs=[pl.BlockSpec((_PAD_BLK, EMB_DIM), lambda i: (i, 0))],
    out_specs=pl.BlockSpec((_PAD_BLK, PAD_DIM), lambda i: (i, 0)),
)(_pad_body)


def _gather_body(idx_hbm, emb1_hbm, emb2_hbm, out1_hbm, out2_hbm,
                 idx_v, rows1_v, rows2_v, sem1, sem2):
    wid = lax.axis_index("s") * _NC + lax.axis_index("c")
    base = wid * _B_PER_W
    for half in range(2):
        cbase = base + half * _HALF
        pltpu.sync_copy(idx_hbm.at[pl.ds(cbase, _HALF)], idx_v)
        c1 = pltpu.async_copy(emb1_hbm.at[idx_v], rows1_v, sem1)
        c2 = pltpu.async_copy(emb2_hbm.at[idx_v], rows2_v, sem2)
        c1.wait()
        pltpu.sync_copy(rows1_v, out1_hbm.at[pl.ds(cbase, _HALF), :])
        c2.wait()
        pltpu.sync_copy(rows2_v, out2_hbm.at[pl.ds(cbase, _HALF), :])


_gather2 = functools.partial(
    pl.kernel,
    mesh=plsc.VectorSubcoreMesh(core_axis_name="c", subcore_axis_name="s"),
    out_type=(
        jax.ShapeDtypeStruct((BATCH, PAD_DIM), jnp.float32),
        jax.ShapeDtypeStruct((BATCH, PAD_DIM), jnp.float32),
    ),
    scratch_types=[
        pltpu.VMEM((_HALF,), jnp.int32),
        pltpu.VMEM((_HALF, PAD_DIM), jnp.float32),
        pltpu.VMEM((_HALF, PAD_DIM), jnp.float32),
        pltpu.SemaphoreType.DMA,
        pltpu.SemaphoreType.DMA,
    ],
)(_gather_body)


def kernel(x_input, emb1, emb2):
    idx = x_input.astype(jnp.int32)
    emb1p = _pad128(emb1)
    emb2p = _pad128(emb2)
    out1, out2 = _gather2(idx, emb1p, emb2p)
    hc = out1[:, :EMB_DIM].reshape(NUM_LAYERS, -1, HIDDEN)
    hx = out2[:, :EMB_DIM].reshape(NUM_LAYERS, -1, HIDDEN)
    return (hc, hx)


# speedup vs baseline: 3.7735x; 1.6218x over previous
import functools

import jax
import jax.numpy as jnp
from jax import lax
from jax.experimental import pallas as pl
from jax.experimental.pallas import tpu as pltpu
from jax.experimental.pallas import tpu_sc as plsc

VOCAB = 1000000
HIDDEN = 32
NUM_LAYERS = 3
BATCH = 16384
EMB_DIM = HIDDEN * NUM_LAYERS

_INFO = plsc.get_sparse_core_info()
_NC = _INFO.num_cores
_NS = _INFO.num_subcores
_NW = _NC * _NS
_B_PER_W = BATCH // _NW
_HALF = _B_PER_W // 2
_NSEM = 8


def _gather_body(idx_hbm, emb1_hbm, emb2_hbm, out1_hbm, out2_hbm,
                 idx_v, rows1_v, rows2_v, sems1, sems2):
    wid = lax.axis_index("s") * _NC + lax.axis_index("c")
    base = wid * _B_PER_W
    pltpu.sync_copy(idx_hbm.at[pl.ds(base, _B_PER_W)], idx_v)

    for half in range(2):
        def issue(v, _):
            vec = idx_v[pl.ds(half * _HALF + v * 16, 16)]
            for j in range(16):
                row = vec[j]
                i = v * 16 + j
                s = j % _NSEM
                pltpu.async_copy(
                    emb1_hbm.at[pl.ds(row, 1), :], rows1_v.at[pl.ds(i, 1), :],
                    sems1.at[s])
                pltpu.async_copy(
                    emb2_hbm.at[pl.ds(row, 1), :], rows2_v.at[pl.ds(i, 1), :],
                    sems2.at[s])
            return ()

        lax.fori_loop(0, _HALF // 16, issue, ())
        per_sem = _HALF // _NSEM
        for s in range(_NSEM):
            pltpu.make_async_copy(
                emb1_hbm.at[pl.ds(0, per_sem), :],
                rows1_v.at[pl.ds(0, per_sem), :], sems1.at[s]).wait()
            pltpu.make_async_copy(
                emb2_hbm.at[pl.ds(0, per_sem), :],
                rows2_v.at[pl.ds(0, per_sem), :], sems2.at[s]).wait()
        out_base = base + half * _HALF
        pltpu.sync_copy(rows1_v, out1_hbm.at[pl.ds(out_base, _HALF), :])
        pltpu.sync_copy(rows2_v, out2_hbm.at[pl.ds(out_base, _HALF), :])


_gather2 = functools.partial(
    pl.kernel,
    mesh=plsc.VectorSubcoreMesh(core_axis_name="c", subcore_axis_name="s"),
    out_type=(
        jax.ShapeDtypeStruct((BATCH, EMB_DIM), jnp.float32),
        jax.ShapeDtypeStruct((BATCH, EMB_DIM), jnp.float32),
    ),
    scratch_types=[
        pltpu.VMEM((_B_PER_W,), jnp.int32),
        pltpu.VMEM((_HALF, EMB_DIM), jnp.float32),
        pltpu.VMEM((_HALF, EMB_DIM), jnp.float32),
        pltpu.SemaphoreType.DMA((_NSEM,)),
        pltpu.SemaphoreType.DMA((_NSEM,)),
    ],
)(_gather_body)


def kernel(x_input, emb1, emb2):
    idx = x_input.astype(jnp.int32)
    out1, out2 = _gather2(idx, emb1, emb2)
    hc = out1.reshape(NUM_LAYERS, -1, HIDDEN)
    hx = out2.reshape(NUM_LAYERS, -1, HIDDEN)
    return (hc, hx)


# speedup vs baseline: 4.0451x; 1.0720x over previous
import functools

import jax
import jax.numpy as jnp
from jax import lax
from jax.experimental import pallas as pl
from jax.experimental.pallas import tpu as pltpu
from jax.experimental.pallas import tpu_sc as plsc

VOCAB = 1000000
HIDDEN = 32
NUM_LAYERS = 3
BATCH = 16384
EMB_DIM = HIDDEN * NUM_LAYERS

_INFO = plsc.get_sparse_core_info()
_NC = _INFO.num_cores
_NS = _INFO.num_subcores
_NW = _NC * _NS
_B_PER_W = BATCH // _NW
_HALF = _B_PER_W // 2
_NSEM = 2


def _gather_body(idx_hbm, emb1_hbm, emb2_hbm, out1_hbm, out2_hbm,
                 idx_v, rows1_v, rows2_v, sems1, sems2):
    wid = lax.axis_index("s") * _NC + lax.axis_index("c")
    base = wid * _B_PER_W
    pltpu.sync_copy(idx_hbm.at[pl.ds(base, _B_PER_W)], idx_v)

    for half in range(2):
        def issue(v, _):
            vec = idx_v[pl.ds(half * _HALF + v * 16, 16)]
            for j in range(16):
                row = vec[j]
                i = v * 16 + j
                s = j % _NSEM
                pltpu.async_copy(
                    emb1_hbm.at[pl.ds(row, 1), :], rows1_v.at[pl.ds(i, 1), :],
                    sems1.at[s])
                pltpu.async_copy(
                    emb2_hbm.at[pl.ds(row, 1), :], rows2_v.at[pl.ds(i, 1), :],
                    sems2.at[s])
            return ()

        lax.fori_loop(0, _HALF // 16, issue, ())
        per_sem = _HALF // _NSEM
        for s in range(_NSEM):
            pltpu.make_async_copy(
                emb1_hbm.at[pl.ds(0, per_sem), :],
                rows1_v.at[pl.ds(0, per_sem), :], sems1.at[s]).wait()
            pltpu.make_async_copy(
                emb2_hbm.at[pl.ds(0, per_sem), :],
                rows2_v.at[pl.ds(0, per_sem), :], sems2.at[s]).wait()
        out_base = base + half * _HALF
        pltpu.sync_copy(rows1_v, out1_hbm.at[pl.ds(out_base, _HALF), :])
        pltpu.sync_copy(rows2_v, out2_hbm.at[pl.ds(out_base, _HALF), :])


_gather2 = functools.partial(
    pl.kernel,
    mesh=plsc.VectorSubcoreMesh(core_axis_name="c", subcore_axis_name="s"),
    out_type=(
        jax.ShapeDtypeStruct((BATCH, EMB_DIM), jnp.float32),
        jax.ShapeDtypeStruct((BATCH, EMB_DIM), jnp.float32),
    ),
    scratch_types=[
        pltpu.VMEM((_B_PER_W,), jnp.int32),
        pltpu.VMEM((_HALF, EMB_DIM), jnp.float32),
        pltpu.VMEM((_HALF, EMB_DIM), jnp.float32),
        pltpu.SemaphoreType.DMA((_NSEM,)),
        pltpu.SemaphoreType.DMA((_NSEM,)),
    ],
)(_gather_body)


def kernel(x_input, emb1, emb2):
    idx = x_input.astype(jnp.int32)
    out1, out2 = _gather2(idx, emb1, emb2)
    hc = out1.reshape(NUM_LAYERS, -1, HIDDEN)
    hx = out2.reshape(NUM_LAYERS, -1, HIDDEN)
    return (hc, hx)
